# baseline (speedup 1.0000x reference)
import functools

import jax
import jax.numpy as jnp
from jax import lax
from jax.experimental import pallas as pl
from jax.experimental.pallas import tpu as pltpu
from jax.experimental.pallas import tpu_sc as plsc

DIM = 2048
N_ROWS = 4 * 8192
NC, NS = 2, 16
NW = NC * NS
RW = N_ROWS // NW
C = 16
NB = 2
S = RW // (NB * C)


def _gather_body(table_hbm, idx_hbm, out_hbm, idx_v, *rest):
    bufs = rest[:NB]
    gsems = rest[NB:2 * NB]
    osems = rest[2 * NB:]

    wid = lax.axis_index("s") * NC + lax.axis_index("c")
    base = wid * RW

    pltpu.sync_copy(idx_hbm.at[pl.ds(base, RW)], idx_v)

    def superstep(s, carry):
        row0 = s * (NB * C)
        gdescs = []
        for b in range(NB):
            @pl.when(s > 0)
            def _drain(b=b):
                pltpu.make_async_copy(
                    bufs[b], out_hbm.at[pl.ds(0, C)], osems[b]).wait()
            gdescs.append(pltpu.async_copy(
                table_hbm.at[idx_v.at[pl.ds(row0 + b * C, C)]],
                bufs[b], gsems[b]))
        for b in range(NB):
            gdescs[b].wait()
            pltpu.async_copy(
                bufs[b], out_hbm.at[pl.ds(base + row0 + b * C, C)], osems[b])
        return carry

    lax.fori_loop(0, S, superstep, 0)

    for b in range(NB):
        pltpu.make_async_copy(bufs[b], out_hbm.at[pl.ds(0, C)], osems[b]).wait()


@functools.lru_cache(maxsize=1)
def _build_sc_gather():
    mesh = plsc.VectorSubcoreMesh(
        core_axis_name="c", subcore_axis_name="s",
        num_cores=NC, num_subcores=NS)
    return pl.kernel(
        _gather_body,
        out_type=jax.ShapeDtypeStruct((N_ROWS, DIM), jnp.float32),
        mesh=mesh,
        scratch_types=(
            [pltpu.VMEM((RW,), jnp.int32)]
            + [pltpu.VMEM((C, DIM), jnp.float32) for _ in range(NB)]
            + [pltpu.SemaphoreType.DMA for _ in range(2 * NB)]
        ),
    )


def kernel(position_ids, pe):
    idx = position_ids.reshape(N_ROWS)
    out = _build_sc_gather()(pe, idx)
    return out.reshape(position_ids.shape + (DIM,))


# speedup vs baseline: 1.0265x; 1.0265x over previous
import functools

import jax
import jax.numpy as jnp
from jax import lax
from jax.experimental import pallas as pl
from jax.experimental.pallas import tpu as pltpu
from jax.experimental.pallas import tpu_sc as plsc

DIM = 2048
N_ROWS = 4 * 8192
NC, NS = 2, 16
NW = NC * NS
RW = N_ROWS // NW
C = 8
NB = 4
S = RW // (NB * C)


def _gather_body(table_hbm, idx_hbm, out_hbm, idx_v, *rest):
    bufs = rest[:NB]
    gsems = rest[NB:2 * NB]
    osems = rest[2 * NB:]

    wid = lax.axis_index("s") * NC + lax.axis_index("c")
    base = wid * RW

    pltpu.sync_copy(idx_hbm.at[pl.ds(base, RW)], idx_v)

    def superstep(s, carry):
        row0 = s * (NB * C)
        gdescs = []
        for b in range(NB):
            @pl.when(s > 0)
            def _drain(b=b):
                pltpu.make_async_copy(
                    bufs[b], out_hbm.at[pl.ds(0, C)], osems[b]).wait()
            gdescs.append(pltpu.async_copy(
                table_hbm.at[idx_v.at[pl.ds(row0 + b * C, C)]],
                bufs[b], gsems[b]))
        for b in range(NB):
            gdescs[b].wait()
            pltpu.async_copy(
                bufs[b], out_hbm.at[pl.ds(base + row0 + b * C, C)], osems[b])
        return carry

    lax.fori_loop(0, S, superstep, 0)

    for b in range(NB):
        pltpu.make_async_copy(bufs[b], out_hbm.at[pl.ds(0, C)], osems[b]).wait()


@functools.lru_cache(maxsize=1)
def _build_sc_gather():
    mesh = plsc.VectorSubcoreMesh(
        core_axis_name="c", subcore_axis_name="s",
        num_cores=NC, num_subcores=NS)
    return pl.kernel(
        _gather_body,
        out_type=jax.ShapeDtypeStruct((N_ROWS, DIM), jnp.float32),
        mesh=mesh,
        scratch_types=(
            [pltpu.VMEM((RW,), jnp.int32)]
            + [pltpu.VMEM((C, DIM), jnp.float32) for _ in range(NB)]
            + [pltpu.SemaphoreType.DMA for _ in range(2 * NB)]
        ),
    )


def kernel(position_ids, pe):
    idx = position_ids.reshape(N_ROWS)
    out = _build_sc_gather()(pe, idx)
    return out.reshape(position_ids.shape + (DIM,))


# speedup vs baseline: 1.0375x; 1.0107x over previous
import functools

import jax
import jax.numpy as jnp
from jax import lax
from jax.experimental import pallas as pl
from jax.experimental.pallas import tpu as pltpu
from jax.experimental.pallas import tpu_sc as plsc

DIM = 2048
N_ROWS = 4 * 8192
NC, NS = 2, 16
NW = NC * NS
RW = N_ROWS // NW
C = 8
NB = 4
S = RW // (NB * C)


def _gather_body(table_hbm, idx_hbm, out_hbm, idx_v, *rest):
    bufs = rest[:NB]
    gsems = rest[NB:2 * NB]
    osems = rest[2 * NB:]

    wid = lax.axis_index("s") * NC + lax.axis_index("c")
    base = wid * RW

    pltpu.sync_copy(idx_hbm.at[pl.ds(base, RW)], idx_v)

    def superstep(t, carry):
        for grp in range(2):
            gdescs = []
            for b in range(2):
                i = grp * 2 + b
                row0 = (t * 4 + grp * 2 + b) * C
                @pl.when(t > 0)
                def _drain(i=i):
                    pltpu.make_async_copy(
                        bufs[i], out_hbm.at[pl.ds(0, C)], osems[i]).wait()
                gdescs.append(pltpu.async_copy(
                    table_hbm.at[idx_v.at[pl.ds(row0, C)]],
                    bufs[i], gsems[i]))
            for b in range(2):
                i = grp * 2 + b
                row0 = (t * 4 + grp * 2 + b) * C
                gdescs[b].wait()
                pltpu.async_copy(
                    bufs[i], out_hbm.at[pl.ds(base + row0, C)], osems[i])
        return carry

    lax.fori_loop(0, S, superstep, 0)

    for b in range(NB):
        pltpu.make_async_copy(bufs[b], out_hbm.at[pl.ds(0, C)], osems[b]).wait()


@functools.lru_cache(maxsize=1)
def _build_sc_gather():
    mesh = plsc.VectorSubcoreMesh(
        core_axis_name="c", subcore_axis_name="s",
        num_cores=NC, num_subcores=NS)
    return pl.kernel(
        _gather_body,
        out_type=jax.ShapeDtypeStruct((N_ROWS, DIM), jnp.float32),
        mesh=mesh,
        scratch_types=(
            [pltpu.VMEM((RW,), jnp.int32)]
            + [pltpu.VMEM((C, DIM), jnp.float32) for _ in range(NB)]
            + [pltpu.SemaphoreType.DMA for _ in range(2 * NB)]
        ),
    )


def kernel(position_ids, pe):
    idx = position_ids.reshape(N_ROWS)
    out = _build_sc_gather()(pe, idx)
    return out.reshape(position_ids.shape + (DIM,))
